# baseline (speedup 1.0000x reference)
import jax
import jax.numpy as jnp
from jax import lax
from jax.experimental import pallas as pl
from jax.experimental.pallas import tpu as pltpu
from jax.experimental.pallas import tpu_sc as plsc

BATCH = 16384
NC = 2
NS = 16
NW = NC * NS
B_PER_W = BATCH // NW
CHUNK = 64
NCHUNK = B_PER_W // CHUNK
LANES = 16


def _irt_body(stu_idx_hbm, exer_idx_hbm, ws_hbm, wk_hbm, we_hbm, out_hbm,
              sidx_v, eidx_v, s_v, k_v, e_v, out_v, osem, *sems):
    wid = lax.axis_index("s") * NC + lax.axis_index("c")
    base = wid * B_PER_W

    idx_copies = [
        (
            pltpu.async_copy(stu_idx_hbm.at[wid, j], sidx_v.at[j], sems[j]),
            pltpu.async_copy(exer_idx_hbm.at[wid, j], eidx_v.at[j], sems[j]),
        )
        for j in range(NCHUNK)
    ]

    gathers = []
    for j in range(NCHUNK):
        for c in idx_copies[j]:
            c.wait()
        gathers.append((
            pltpu.async_copy(ws_hbm.at[sidx_v.at[j]], s_v.at[j], sems[j]),
            pltpu.async_copy(wk_hbm.at[eidx_v.at[j]], k_v.at[j], sems[j]),
            pltpu.async_copy(we_hbm.at[eidx_v.at[j]], e_v.at[j], sems[j]),
        ))

    one = jnp.full((LANES,), 1.0, dtype=jnp.float32)
    out_copies = []
    for j in range(NCHUNK):
        for c in gathers[j]:
            c.wait()
        for i in range(CHUNK // LANES):
            sl = pl.ds(i * LANES, LANES)
            es = jnp.exp(-s_v[j, sl])
            ek = jnp.exp(-k_v[j, sl])
            ee = jnp.exp(-e_v[j, sl])
            z = (1.7 * (ek - es)) / ((one + ee) * (one + es) * (one + ek))
            out_v[pl.ds(j * CHUNK + i * LANES, LANES)] = one / (one + jnp.exp(-z))
        out_copies.append(pltpu.async_copy(
            out_v.at[pl.ds(j * CHUNK, CHUNK)],
            out_hbm.at[pl.ds(base + j * CHUNK, CHUNK)], osem))
    for c in out_copies:
        c.wait()


@jax.jit
def _irt_sc(stu_idx, exer_idx, ws, wk, we):
    mesh = plsc.VectorSubcoreMesh(core_axis_name="c", subcore_axis_name="s")
    return pl.kernel(
        _irt_body,
        mesh=mesh,
        out_type=jax.ShapeDtypeStruct((BATCH,), jnp.float32),
        scratch_types=[
            pltpu.VMEM((NCHUNK, CHUNK), jnp.int32),
            pltpu.VMEM((NCHUNK, CHUNK), jnp.int32),
            pltpu.VMEM((NCHUNK, CHUNK), jnp.float32),
            pltpu.VMEM((NCHUNK, CHUNK), jnp.float32),
            pltpu.VMEM((NCHUNK, CHUNK), jnp.float32),
            pltpu.VMEM((B_PER_W,), jnp.float32),
            pltpu.SemaphoreType.DMA,
        ] + [pltpu.SemaphoreType.DMA] * NCHUNK,
    )(stu_idx, exer_idx, ws, wk, we)


def kernel(stu_id, exer_id, W_student, W_k_difficulty, W_e_discrimination):
    stu_idx = stu_id.astype(jnp.int32).reshape(NW, NCHUNK, CHUNK)
    exer_idx = exer_id.astype(jnp.int32).reshape(NW, NCHUNK, CHUNK)
    ws = W_student.reshape(-1)
    wk = W_k_difficulty.reshape(-1)
    we = W_e_discrimination.reshape(-1)
    return _irt_sc(stu_idx, exer_idx, ws, wk, we)


# speedup vs baseline: 1.0425x; 1.0425x over previous
import jax
import jax.numpy as jnp
from jax import lax
from jax.experimental import pallas as pl
from jax.experimental.pallas import tpu as pltpu
from jax.experimental.pallas import tpu_sc as plsc

BATCH = 16384
NC = 2
NS = 16
NW = NC * NS
B_PER_W = BATCH // NW
CHUNK = 128
NCHUNK = B_PER_W // CHUNK
LANES = 16


def _irt_body(stu_idx_hbm, exer_idx_hbm, ws_hbm, wk_hbm, we_hbm, out_hbm,
              sidx_v, eidx_v, s_v, k_v, e_v, out_v, osem, *sems):
    wid = lax.axis_index("s") * NC + lax.axis_index("c")
    base = wid * B_PER_W

    idx_copies = [
        (
            pltpu.async_copy(stu_idx_hbm.at[wid, j], sidx_v.at[j], sems[j]),
            pltpu.async_copy(exer_idx_hbm.at[wid, j], eidx_v.at[j], sems[j]),
        )
        for j in range(NCHUNK)
    ]

    gathers = []
    with jax.named_scope("idx_wait_fire"):
        for j in range(NCHUNK):
            for c in idx_copies[j]:
                c.wait()
            gathers.append((
                pltpu.async_copy(ws_hbm.at[sidx_v.at[j]], s_v.at[j], sems[j]),
                pltpu.async_copy(wk_hbm.at[eidx_v.at[j]], k_v.at[j], sems[j]),
                pltpu.async_copy(we_hbm.at[eidx_v.at[j]], e_v.at[j], sems[j]),
            ))

    one = jnp.full((LANES,), 1.0, dtype=jnp.float32)
    out_copies = []
    for j in range(NCHUNK):
        with jax.named_scope(f"gwait{j}"):
            for c in gathers[j]:
                c.wait()
        with jax.named_scope(f"compute{j}"):
            for i in range(CHUNK // LANES):
                sl = pl.ds(i * LANES, LANES)
                es = jnp.exp(-s_v[j, sl])
                ek = jnp.exp(-k_v[j, sl])
                ee = jnp.exp(-e_v[j, sl])
                z = (1.7 * (ek - es)) / ((one + ee) * (one + es) * (one + ek))
                out_v[pl.ds(j * CHUNK + i * LANES, LANES)] = one / (one + jnp.exp(-z))
            out_copies.append(pltpu.async_copy(
                out_v.at[pl.ds(j * CHUNK, CHUNK)],
                out_hbm.at[pl.ds(base + j * CHUNK, CHUNK)], osem))
    with jax.named_scope("out_wait"):
        for c in out_copies:
            c.wait()


@jax.jit
def _irt_sc(stu_idx, exer_idx, ws, wk, we):
    mesh = plsc.VectorSubcoreMesh(core_axis_name="c", subcore_axis_name="s")
    return pl.kernel(
        _irt_body,
        mesh=mesh,
        out_type=jax.ShapeDtypeStruct((BATCH,), jnp.float32),
        scratch_types=[
            pltpu.VMEM((NCHUNK, CHUNK), jnp.int32),
            pltpu.VMEM((NCHUNK, CHUNK), jnp.int32),
            pltpu.VMEM((NCHUNK, CHUNK), jnp.float32),
            pltpu.VMEM((NCHUNK, CHUNK), jnp.float32),
            pltpu.VMEM((NCHUNK, CHUNK), jnp.float32),
            pltpu.VMEM((B_PER_W,), jnp.float32),
            pltpu.SemaphoreType.DMA,
        ] + [pltpu.SemaphoreType.DMA] * NCHUNK,
    )(stu_idx, exer_idx, ws, wk, we)


def kernel(stu_id, exer_id, W_student, W_k_difficulty, W_e_discrimination):
    stu_idx = stu_id.astype(jnp.int32).reshape(NW, NCHUNK, CHUNK)
    exer_idx = exer_id.astype(jnp.int32).reshape(NW, NCHUNK, CHUNK)
    ws = W_student.reshape(-1)
    wk = W_k_difficulty.reshape(-1)
    we = W_e_discrimination.reshape(-1)
    return _irt_sc(stu_idx, exer_idx, ws, wk, we)


# speedup vs baseline: 1.0468x; 1.0041x over previous
import jax
import jax.numpy as jnp
from jax import lax
from jax.experimental import pallas as pl
from jax.experimental.pallas import tpu as pltpu
from jax.experimental.pallas import tpu_sc as plsc

BATCH = 16384
NC = 2
NS = 16
NW = NC * NS
B_PER_W = BATCH // NW
CHUNK = 128
NCHUNK = B_PER_W // CHUNK
LANES = 16
WAVES = [(0, 0, 64), (0, 64, 64), (1, 0, 128), (2, 0, 128), (3, 0, 128)]


def _irt_body(stu_idx_hbm, exer_idx_hbm, ws_hbm, wk_hbm, we_hbm, out_hbm,
              sidx_v, eidx_v, s_v, k_v, e_v, out_v, osem, *sems):
    isems = sems[:NCHUNK]
    gsems = sems[NCHUNK:]
    wid = lax.axis_index("s") * NC + lax.axis_index("c")
    base = wid * B_PER_W

    idx_copies = [
        (
            pltpu.async_copy(stu_idx_hbm.at[wid, j], sidx_v.at[j], isems[j]),
            pltpu.async_copy(exer_idx_hbm.at[wid, j], eidx_v.at[j], isems[j]),
        )
        for j in range(NCHUNK)
    ]

    gathers = []
    waited_rows = set()
    for w, (j, off, ln) in enumerate(WAVES):
        if j not in waited_rows:
            for c in idx_copies[j]:
                c.wait()
            waited_rows.add(j)
        sl = pl.ds(off, ln)
        gathers.append((
            pltpu.async_copy(ws_hbm.at[sidx_v.at[j, sl]], s_v.at[j, sl], gsems[w]),
            pltpu.async_copy(wk_hbm.at[eidx_v.at[j, sl]], k_v.at[j, sl], gsems[w]),
            pltpu.async_copy(we_hbm.at[eidx_v.at[j, sl]], e_v.at[j, sl], gsems[w]),
        ))

    one = jnp.full((LANES,), 1.0, dtype=jnp.float32)
    out_copies = []
    for w, (j, off, ln) in enumerate(WAVES):
        for c in gathers[w]:
            c.wait()
        for i in range(ln // LANES):
            sl = pl.ds(off + i * LANES, LANES)
            es = jnp.exp(-s_v[j, sl])
            ek = jnp.exp(-k_v[j, sl])
            ee = jnp.exp(-e_v[j, sl])
            z = (1.7 * (ek - es)) / ((one + ee) * (one + es) * (one + ek))
            out_v[pl.ds(j * CHUNK + off + i * LANES, LANES)] = one / (one + jnp.exp(-z))
        out_copies.append(pltpu.async_copy(
            out_v.at[pl.ds(j * CHUNK + off, ln)],
            out_hbm.at[pl.ds(base + j * CHUNK + off, ln)], osem))
    for c in out_copies:
        c.wait()


@jax.jit
def _irt_sc(stu_idx, exer_idx, ws, wk, we):
    mesh = plsc.VectorSubcoreMesh(core_axis_name="c", subcore_axis_name="s")
    return pl.kernel(
        _irt_body,
        mesh=mesh,
        out_type=jax.ShapeDtypeStruct((BATCH,), jnp.float32),
        scratch_types=[
            pltpu.VMEM((NCHUNK, CHUNK), jnp.int32),
            pltpu.VMEM((NCHUNK, CHUNK), jnp.int32),
            pltpu.VMEM((NCHUNK, CHUNK), jnp.float32),
            pltpu.VMEM((NCHUNK, CHUNK), jnp.float32),
            pltpu.VMEM((NCHUNK, CHUNK), jnp.float32),
            pltpu.VMEM((B_PER_W,), jnp.float32),
            pltpu.SemaphoreType.DMA,
        ] + [pltpu.SemaphoreType.DMA] * (NCHUNK + len(WAVES)),
    )(stu_idx, exer_idx, ws, wk, we)


def kernel(stu_id, exer_id, W_student, W_k_difficulty, W_e_discrimination):
    stu_idx = stu_id.astype(jnp.int32).reshape(NW, NCHUNK, CHUNK)
    exer_idx = exer_id.astype(jnp.int32).reshape(NW, NCHUNK, CHUNK)
    ws = W_student.reshape(-1)
    wk = W_k_difficulty.reshape(-1)
    we = W_e_discrimination.reshape(-1)
    return _irt_sc(stu_idx, exer_idx, ws, wk, we)
